# Initial kernel scaffold; baseline (speedup 1.0000x reference)
import jax
import jax.numpy as jnp
from jax.experimental import pallas as pl


def kernel(anchors, objectness, bbox_deltas):
    raise NotImplementedError("write your pallas kernel here")


# speedup vs baseline: 202.5615x; 202.5615x over previous
import functools
import math

import jax
import jax.numpy as jnp
from jax import lax
from jax.experimental import pallas as pl

H_IMG, W_IMG = 800.0, 1216.0
PRE = 2000
NPAD = 2048
POST = 1000
TH = 0.7
MIN_SZ = 1.0
CLIP = math.log(1000.0 / 16.0)
BLK = 256
NBLK = NPAD // BLK


def _decode(anc4, dl4):
    x1, y1, x2, y2 = anc4
    dx, dy, dw, dh = dl4
    w = x2 - x1
    h = y2 - y1
    cx = x1 + 0.5 * w
    cy = y1 + 0.5 * h
    dwc = jnp.minimum(dw, CLIP)
    dhc = jnp.minimum(dh, CLIP)
    pcx = dx * w + cx
    pcy = dy * h + cy
    pw = jnp.exp(dwc) * w
    ph = jnp.exp(dhc) * h
    px1 = jnp.clip(pcx - 0.5 * pw, 0.0, W_IMG)
    py1 = jnp.clip(pcy - 0.5 * ph, 0.0, H_IMG)
    px2 = jnp.clip(pcx + 0.5 * pw, 0.0, W_IMG)
    py2 = jnp.clip(pcy + 0.5 * ph, 0.0, H_IMG)
    return px1, py1, px2, py2


def _nms_body(anc_c_ref, del_c_ref, anc_r_ref, del_r_ref, sc_ref,
              fs_ref, box_ref):
    ar = tuple(anc_r_ref[0, k:k + 1, :] for k in range(4))
    dr = tuple(del_r_ref[0, k:k + 1, :] for k in range(4))
    rx1, ry1, rx2, ry2 = _decode(ar, dr)
    area_r = (rx2 - rx1) * (ry2 - ry1)

    col = lax.broadcasted_iota(jnp.int32, (1, NPAD), 1)
    real = col < PRE
    valid = (rx2 - rx1 >= MIN_SZ) & (ry2 - ry1 >= MIN_SZ) & real
    keep0 = valid.astype(jnp.float32)

    def block_step(r0, keep):
        anc_blk = anc_c_ref[0, r0:r0 + BLK, :]
        del_blk = del_c_ref[0, r0:r0 + BLK, :]
        bx1, by1, bx2, by2 = _decode(
            tuple(anc_blk[:, k:k + 1] for k in range(4)),
            tuple(del_blk[:, k:k + 1] for k in range(4)))
        area_b = (bx2 - bx1) * (by2 - by1)
        iw = jnp.maximum(jnp.minimum(bx2, rx2) - jnp.maximum(bx1, rx1), 0.0)
        ih = jnp.maximum(jnp.minimum(by2, ry2) - jnp.maximum(by1, ry1), 0.0)
        inter = iw * ih
        iou = inter / (area_b + area_r - inter + 1e-9)
        rowid = r0 + lax.broadcasted_iota(jnp.int32, (BLK, 1), 0)
        suppf = ((iou > TH) & (col > rowid)).astype(jnp.float32)
        sblk = suppf[:, r0:r0 + BLK]
        kb0 = keep[:, r0:r0 + BLK]

        def fix_cond(c):
            return c[1]

        def fix_body(c):
            kb, _ = c
            cnt = jnp.dot(kb, sblk, preferred_element_type=jnp.float32)
            kb2 = kb0 * (cnt < 0.5).astype(jnp.float32)
            return kb2, jnp.sum(jnp.abs(kb2 - kb)) > 0.0

        kb, _ = lax.while_loop(fix_cond, fix_body, (kb0, jnp.asarray(True)))
        cnt_all = jnp.dot(kb, suppf, preferred_element_type=jnp.float32)
        return keep * (cnt_all < 0.5).astype(jnp.float32)

    keep = keep0
    for b in range(NBLK):
        keep = block_step(b * BLK, keep)

    neg = jnp.where(real, -1e9, -2e9)
    fs_ref[0, 0:1, :] = jnp.where(keep > 0.5, sc_ref[0, 0:1, :], neg)
    box_ref[0, 0:1, :] = rx1
    box_ref[0, 1:2, :] = ry1
    box_ref[0, 2:3, :] = rx2
    box_ref[0, 3:4, :] = ry2


@functools.partial(jax.jit)
def kernel(anchors, objectness, bbox_deltas):
    B = objectness.shape[0]
    top_vals, top_idx = lax.top_k(objectness, PRE)
    anc = jnp.take(anchors, top_idx, axis=0)
    dl = jnp.take_along_axis(bbox_deltas, top_idx[..., None], axis=1)
    pad = NPAD - PRE
    anc_c = jnp.pad(anc, ((0, 0), (0, pad), (0, 0)))
    del_c = jnp.pad(dl, ((0, 0), (0, pad), (0, 0)))
    sc = jnp.pad(top_vals, ((0, 0), (0, pad)))[:, None, :]
    anc_r = anc_c.transpose(0, 2, 1)
    del_r = del_c.transpose(0, 2, 1)

    fs, box_r = pl.pallas_call(
        _nms_body,
        grid=(B,),
        in_specs=[
            pl.BlockSpec((1, NPAD, 4), lambda b: (b, 0, 0)),
            pl.BlockSpec((1, NPAD, 4), lambda b: (b, 0, 0)),
            pl.BlockSpec((1, 4, NPAD), lambda b: (b, 0, 0)),
            pl.BlockSpec((1, 4, NPAD), lambda b: (b, 0, 0)),
            pl.BlockSpec((1, 1, NPAD), lambda b: (b, 0, 0)),
        ],
        out_specs=[
            pl.BlockSpec((1, 1, NPAD), lambda b: (b, 0, 0)),
            pl.BlockSpec((1, 4, NPAD), lambda b: (b, 0, 0)),
        ],
        out_shape=[
            jax.ShapeDtypeStruct((B, 1, NPAD), jnp.float32),
            jax.ShapeDtypeStruct((B, 4, NPAD), jnp.float32),
        ],
    )(anc_c, del_c, anc_r, del_r, sc)

    boxes = box_r.transpose(0, 2, 1)
    _, sel = lax.top_k(fs[:, 0, :], POST)
    return jnp.take_along_axis(boxes, sel[..., None], axis=1)
